# baseline (speedup 1.0000x reference)
import functools

import jax
import jax.numpy as jnp
from jax import lax
from jax.experimental import pallas as pl
from jax.experimental.pallas import tpu as pltpu
from jax.experimental.pallas import tpu_sc as plsc

_NC = 2
_NS = 16
_NW = _NC * _NS
_L = 16
_G = 64


def _mesh():
    return plsc.VectorSubcoreMesh(core_axis_name="c", subcore_axis_name="s",
                                  num_cores=_NC, num_subcores=_NS)


def _pad_rows(N):
    return -(-N // (8 * _NS)) * (8 * _NS)


def _tc_deg(dst_row, dst_col, N, EB):
    E = dst_row.shape[1]
    HI = -(-N // 128)
    grid = E // EB

    def body(dr_ref, dc_ref, o_ref):
        i = pl.program_id(0)

        @pl.when(i == 0)
        def _init():
            o_ref[...] = jnp.zeros_like(o_ref)

        hi = dr_ref[...] // 128
        lo = dc_ref[...] % 128
        oh_hi = (lax.broadcasted_iota(jnp.int32, (HI, 1), 0) == hi
                 ).astype(jnp.float32)
        oh_lo = (lo == lax.broadcasted_iota(jnp.int32, (1, 128), 1)
                 ).astype(jnp.float32)
        o_ref[...] += jnp.dot(oh_hi, oh_lo,
                              preferred_element_type=jnp.float32)

        @pl.when(i == grid - 1)
        def _finish():
            o_ref[...] = lax.rsqrt(o_ref[...] + 1.0)

    return pl.pallas_call(
        body,
        grid=(grid,),
        in_specs=[pl.BlockSpec((1, EB), lambda i: (0, i)),
                  pl.BlockSpec((EB, 1), lambda i: (i, 0))],
        out_specs=pl.BlockSpec((HI, 128), lambda i: (0, 0)),
        out_shape=jax.ShapeDtypeStruct((HI, 128), jnp.float32),
    )(dst_row, dst_col)


@functools.lru_cache(maxsize=None)
def _make_agg_kernel(N, F, K, NCHUNK):
    NP = _pad_rows(N)
    RT = NP // _NS
    NB, NQ = 4, 8
    NG = NCHUNK // NQ
    assert NCHUNK % NQ == 0

    @functools.partial(
        pl.kernel,
        out_type=jax.ShapeDtypeStruct((_NC, NP, F), jnp.float32),
        mesh=_mesh(),
        scratch_types=(
            [pltpu.VMEM_SHARED((NP, F), jnp.float32)]
            + [pltpu.VMEM((K, F), jnp.float32) for _ in range(NB)]
            + [pltpu.VMEM((NQ, 2, K), jnp.int32)]
            + [pltpu.SemaphoreType.DMA] * (2 * NB + NQ)
        ),
    )
    def agg_kernel(y_hbm, e_hbm, z_hbm, out_hbm, acc, *rest):
        rows = rest[:NB]
        slots = rest[NB]
        gsem = rest[NB + 1: NB + 1 + NB]
        ssem = rest[NB + 1 + NB: NB + 1 + 2 * NB]
        isem = rest[NB + 1 + 2 * NB:]
        c_ax = lax.axis_index("c")
        s_ax = lax.axis_index("s")
        wid = s_ax * _NC + c_ax

        pltpu.sync_copy(z_hbm.at[pl.ds(s_ax * RT, RT), :],
                        acc.at[pl.ds(s_ax * RT, RT), :])
        plsc.subcore_barrier()

        def idx_load(ci, q):
            return pltpu.async_copy(e_hbm.at[wid, ci], slots.at[q], isem[q])

        def gather(q, b):
            return pltpu.async_copy(y_hbm.at[slots.at[q, 0]], rows[b], gsem[b])

        def scatter(q, b):
            return pltpu.async_copy(rows[b], acc.at[slots.at[q, 1]], ssem[b],
                                    add=True)

        for q in range(NQ):
            idx_load(q, q)
        pltpu.make_async_copy(e_hbm.at[wid, 0], slots.at[0], isem[0]).wait()
        gather(0, 0)
        pltpu.make_async_copy(e_hbm.at[wid, 1], slots.at[1], isem[1]).wait()
        gather(1, 1)

        def group(g, carry):
            for k in range(NQ):
                b, q = k % NB, k
                b2, q2 = (k - 2) % NB, (k - 2) % NQ
                pltpu.make_async_copy(y_hbm.at[slots.at[q, 0]], rows[b],
                                      gsem[b]).wait()
                scatter(q, b)

                def _wait_sc():
                    pltpu.make_async_copy(rows[b2], acc.at[slots.at[q2, 1]],
                                          ssem[b2]).wait()
                if k >= 2:
                    _wait_sc()
                else:
                    pl.when(g > 0)(_wait_sc)

                def _iload():
                    idx_load(g * NQ + k + 6, q2)
                if k >= 2:
                    pl.when(g * NQ + k + 6 < NCHUNK)(_iload)
                else:
                    pl.when(jnp.logical_and(
                        g > 0, g * NQ + k + 6 < NCHUNK))(_iload)

                q3 = (k + 2) % NQ
                def _gath():
                    pltpu.make_async_copy(e_hbm.at[wid, g * NQ + k + 2],
                                          slots.at[q3], isem[q3]).wait()
                    gather(q3, b2)
                if k < NQ - 2:
                    _gath()
                else:
                    pl.when(g < NG - 1)(_gath)
            return carry

        lax.fori_loop(0, NG, group, 0)

        for k in (NQ - 2, NQ - 1):
            pltpu.make_async_copy(rows[k % NB], acc.at[slots.at[k, 1]],
                                  ssem[k % NB]).wait()

        plsc.subcore_barrier()
        pltpu.sync_copy(acc.at[pl.ds(s_ax * RT, RT), :],
                        out_hbm.at[c_ax, pl.ds(s_ax * RT, RT), :])

    return agg_kernel


def _prep_edges(src, dst, N, K, NCHUNK):
    E = src.shape[0]
    pad = _NW * NCHUNK * K - E
    src_p = jnp.concatenate([src, jnp.zeros((pad,), src.dtype)])
    dst_p = jnp.concatenate([dst, jnp.full((pad,), N, dst.dtype)])
    return jnp.concatenate(
        [src_p.reshape(_NW, NCHUNK, 1, K), dst_p.reshape(_NW, NCHUNK, 1, K)],
        axis=2)


def _tc_first(dis, x, W):
    N, F = x.shape
    H = W.shape[1]

    def body(dis_ref, x_ref, w_ref, y_ref):
        xw = jnp.dot(x_ref[...], w_ref[...],
                     preferred_element_type=jnp.float32,
                     precision=lax.Precision.HIGHEST)
        y_ref[...] = dis_ref[...] * xw

    return pl.pallas_call(
        body,
        out_shape=jax.ShapeDtypeStruct((N, H), jnp.float32),
    )(dis, x, W)


def _tc_mid(sp, y, dis, b, W):
    N, H = y.shape

    def body(sp_ref, y_ref, dis_ref, b_ref, w_ref, o_ref):
        sagg = sp_ref[0][:N] + sp_ref[1][:N] + y_ref[...]
        h = jnp.maximum(dis_ref[...] * sagg + b_ref[...], 0.0)
        hw = jnp.dot(h, w_ref[...],
                     preferred_element_type=jnp.float32,
                     precision=lax.Precision.HIGHEST)
        o_ref[...] = dis_ref[...] * hw

    return pl.pallas_call(
        body,
        out_shape=jax.ShapeDtypeStruct((N, W.shape[1]), jnp.float32),
    )(sp, y, dis, b, W)


def _tc_final(sp, y, dis, b, batch2d, Wh1, bh1, Wh2, bh2):
    N, H = y.shape

    def body(sp_ref, y_ref, dis_ref, b_ref, bt_ref,
             w1_ref, b1_ref, w2_ref, b2_ref, o_ref):
        h2 = jnp.maximum(
            dis_ref[...] * (sp_ref[0][:N] + sp_ref[1][:N] + y_ref[...])
            + b_ref[...],
            0.0)
        gids = lax.broadcasted_iota(jnp.int32, (_G, 1), 0)
        oh = (gids == bt_ref[...]).astype(jnp.float32)
        cnt = jnp.sum(oh, axis=1, keepdims=True)
        summ = jnp.dot(oh, h2,
                       preferred_element_type=jnp.float32,
                       precision=lax.Precision.HIGHEST)
        pooled = summ / jnp.maximum(cnt, 1.0)
        t = jnp.maximum(
            jnp.dot(pooled, w1_ref[...],
                    preferred_element_type=jnp.float32,
                    precision=lax.Precision.HIGHEST) + b1_ref[...],
            0.0)
        o_ref[...] = jnp.dot(t, w2_ref[...],
                             preferred_element_type=jnp.float32,
                             precision=lax.Precision.HIGHEST) + b2_ref[...]

    return pl.pallas_call(
        body,
        out_shape=jax.ShapeDtypeStruct((_G, 1), jnp.float32),
    )(sp, y, dis, b, batch2d, Wh1, bh1, Wh2, bh2)


def kernel(x, edge_index, batch, W1, b1, W2, b2, Wh1, bh1, Wh2, bh2):
    N, F = x.shape
    E = edge_index.shape[1]
    K = 64
    NCHUNK = -(-E // (_NW * K * 8)) * 8

    src = edge_index[0]
    dst = edge_index[1]

    dis_mat = _tc_deg(dst.reshape(1, E), dst.reshape(E, 1), N, 6400)
    dis = dis_mat.reshape(-1, 1)[:N]
    y1 = _tc_first(dis, x.astype(jnp.float32), W1)
    agg = _make_agg_kernel(N, F, K, NCHUNK)
    e_il = _prep_edges(src, dst, N, K, NCHUNK)
    z = jnp.zeros((_pad_rows(N), F), jnp.float32)
    s1p = agg(y1, e_il, z)
    y2 = _tc_mid(s1p, y1, dis, b1, W2)
    s2p = agg(y2, e_il, z)
    out = _tc_final(s2p, y2, dis, b2,
                    batch.reshape(1, N).astype(jnp.int32),
                    Wh1, bh1, Wh2, bh2)
    return out


# speedup vs baseline: 1.0668x; 1.0668x over previous
import functools

import jax
import jax.numpy as jnp
from jax import lax
from jax.experimental import pallas as pl
from jax.experimental.pallas import tpu as pltpu
from jax.experimental.pallas import tpu_sc as plsc

_NC = 2
_NS = 16
_NW = _NC * _NS
_L = 16
_G = 64


def _mesh():
    return plsc.VectorSubcoreMesh(core_axis_name="c", subcore_axis_name="s",
                                  num_cores=_NC, num_subcores=_NS)


def _pad_rows(N):
    return -(-N // (8 * _NS)) * (8 * _NS)


def _tc_deg(dst_row, dst_col, N, EB):
    E = dst_row.shape[1]
    HI = -(-N // 128)
    grid = E // EB

    def body(dr_ref, dc_ref, o_ref):
        i = pl.program_id(0)

        @pl.when(i == 0)
        def _init():
            o_ref[...] = jnp.zeros_like(o_ref)

        hi = dr_ref[...] // 128
        lo = dc_ref[...] % 128
        oh_hi = (lax.broadcasted_iota(jnp.int32, (HI, 1), 0) == hi
                 ).astype(jnp.float32)
        oh_lo = (lo == lax.broadcasted_iota(jnp.int32, (1, 128), 1)
                 ).astype(jnp.float32)
        o_ref[...] += jnp.dot(oh_hi, oh_lo,
                              preferred_element_type=jnp.float32)

        @pl.when(i == grid - 1)
        def _finish():
            o_ref[...] = lax.rsqrt(o_ref[...] + 1.0)

    return pl.pallas_call(
        body,
        grid=(grid,),
        in_specs=[pl.BlockSpec((1, EB), lambda i: (0, i)),
                  pl.BlockSpec((EB, 1), lambda i: (i, 0))],
        out_specs=pl.BlockSpec((HI, 128), lambda i: (0, 0)),
        out_shape=jax.ShapeDtypeStruct((HI, 128), jnp.float32),
    )(dst_row, dst_col)


@functools.lru_cache(maxsize=None)
def _make_agg_kernel(N, F, K, NCHUNK):
    NP = _pad_rows(N)
    RT = NP // _NS
    NB, NQ = 4, 8
    NG = NCHUNK // NQ
    assert NCHUNK % NQ == 0

    @functools.partial(
        pl.kernel,
        out_type=jax.ShapeDtypeStruct((_NC, NP, F), jnp.float32),
        mesh=_mesh(),
        scratch_types=(
            [pltpu.VMEM_SHARED((NP, F), jnp.float32)]
            + [pltpu.VMEM((K, F), jnp.float32) for _ in range(NB)]
            + [pltpu.VMEM((NQ, 2, K), jnp.int32)]
            + [pltpu.SemaphoreType.DMA] * (2 * NB + NQ)
        ),
    )
    def agg_kernel(y_hbm, e_hbm, z_hbm, out_hbm, acc, *rest):
        rows = rest[:NB]
        slots = rest[NB]
        gsem = rest[NB + 1: NB + 1 + NB]
        ssem = rest[NB + 1 + NB: NB + 1 + 2 * NB]
        isem = rest[NB + 1 + 2 * NB:]
        c_ax = lax.axis_index("c")
        s_ax = lax.axis_index("s")
        wid = s_ax * _NC + c_ax

        pltpu.sync_copy(z_hbm.at[pl.ds(s_ax * RT, RT), :],
                        acc.at[pl.ds(s_ax * RT, RT), :])
        plsc.subcore_barrier()

        def idx_load(ci, q):
            return pltpu.async_copy(e_hbm.at[wid, ci], slots.at[q], isem[q])

        def gather(q, b):
            return pltpu.async_copy(y_hbm.at[slots.at[q, 0]], rows[b], gsem[b])

        def scatter(q, b):
            return pltpu.async_copy(rows[b], acc.at[slots.at[q, 1]], ssem[b],
                                    add=True)

        for q in range(NQ):
            idx_load(q, q)
        pltpu.make_async_copy(e_hbm.at[wid, 0], slots.at[0], isem[0]).wait()
        gather(0, 0)
        pltpu.make_async_copy(e_hbm.at[wid, 1], slots.at[1], isem[1]).wait()
        gather(1, 1)

        def group(g, carry):
            for k in range(NQ):
                b, q = k % NB, k
                b2, q2 = (k - 2) % NB, (k - 2) % NQ
                pltpu.make_async_copy(y_hbm.at[slots.at[q, 0]], rows[b],
                                      gsem[b]).wait()
                scatter(q, b)

                def _wait_sc():
                    pltpu.make_async_copy(rows[b2], acc.at[slots.at[q2, 1]],
                                          ssem[b2]).wait()
                if k >= 2:
                    _wait_sc()
                else:
                    pl.when(g > 0)(_wait_sc)

                def _iload():
                    idx_load(g * NQ + k + 6, q2)
                if k >= 2:
                    pl.when(g * NQ + k + 6 < NCHUNK)(_iload)
                else:
                    pl.when(jnp.logical_and(
                        g > 0, g * NQ + k + 6 < NCHUNK))(_iload)

                q3 = (k + 2) % NQ
                def _gath():
                    pltpu.make_async_copy(e_hbm.at[wid, g * NQ + k + 2],
                                          slots.at[q3], isem[q3]).wait()
                    gather(q3, b2)
                if k < NQ - 2:
                    _gath()
                else:
                    pl.when(g < NG - 1)(_gath)
            return carry

        lax.fori_loop(0, NG, group, 0)

        for k in (NQ - 2, NQ - 1):
            pltpu.make_async_copy(rows[k % NB], acc.at[slots.at[k, 1]],
                                  ssem[k % NB]).wait()

        plsc.subcore_barrier()
        pltpu.sync_copy(acc.at[pl.ds(s_ax * RT, RT), :],
                        out_hbm.at[c_ax, pl.ds(s_ax * RT, RT), :])

    return agg_kernel


@functools.lru_cache(maxsize=None)
def _make_agg_serial(N, F, K, NCHUNK):
    NP = _pad_rows(N)
    RT = NP // _NS

    @functools.partial(
        pl.kernel,
        out_type=jax.ShapeDtypeStruct((_NC, NP, F), jnp.float32),
        mesh=_mesh(),
        scratch_types=[
            pltpu.VMEM_SHARED((NP, F), jnp.float32),
            pltpu.VMEM((2, K), jnp.int32),
            pltpu.VMEM((K, F), jnp.float32),
            pltpu.SemaphoreType.DMA,
        ],
    )
    def agg_kernel(y_hbm, e_hbm, z_hbm, out_hbm, acc, slot, rows, sem):
        c_ax = lax.axis_index("c")
        s_ax = lax.axis_index("s")
        wid = s_ax * _NC + c_ax

        pltpu.sync_copy(z_hbm.at[pl.ds(s_ax * RT, RT), :],
                        acc.at[pl.ds(s_ax * RT, RT), :])
        plsc.subcore_barrier()

        def chunk(ci, carry):
            pltpu.sync_copy(e_hbm.at[wid, ci], slot)
            pltpu.async_copy(y_hbm.at[slot.at[0]], rows, sem).wait()
            pltpu.sync_copy(rows, acc.at[slot.at[1]], add=True)
            return carry
        lax.fori_loop(0, NCHUNK, chunk, 0)

        plsc.subcore_barrier()
        pltpu.sync_copy(acc.at[pl.ds(s_ax * RT, RT), :],
                        out_hbm.at[c_ax, pl.ds(s_ax * RT, RT), :])

    return agg_kernel


@functools.lru_cache(maxsize=None)
def _make_agg_pair(N, F, K, NCHUNK):
    NP = _pad_rows(N)
    RT = NP // _NS
    NPAIR = NCHUNK // 2
    assert NCHUNK % 2 == 0

    @functools.partial(
        pl.kernel,
        out_type=jax.ShapeDtypeStruct((_NC, NP, F), jnp.float32),
        mesh=_mesh(),
        scratch_types=[
            pltpu.VMEM_SHARED((NP, F), jnp.float32),
            pltpu.VMEM((2, 2, K), jnp.int32),
            pltpu.VMEM((K, F), jnp.float32),
            pltpu.VMEM((K, F), jnp.float32),
            pltpu.SemaphoreType.DMA,
            pltpu.SemaphoreType.DMA,
            pltpu.SemaphoreType.DMA,
            pltpu.SemaphoreType.DMA,
        ],
    )
    def agg_kernel(y_hbm, e_hbm, z_hbm, out_hbm,
                   acc, slots, rows0, rows1, gsem0, gsem1, isem0, isem1):
        c_ax = lax.axis_index("c")
        s_ax = lax.axis_index("s")
        wid = s_ax * _NC + c_ax

        pltpu.sync_copy(z_hbm.at[pl.ds(s_ax * RT, RT), :],
                        acc.at[pl.ds(s_ax * RT, RT), :])
        plsc.subcore_barrier()

        pltpu.sync_copy(e_hbm.at[wid, 0], slots.at[0])
        pltpu.sync_copy(e_hbm.at[wid, 1], slots.at[1])
        pltpu.async_copy(y_hbm.at[slots.at[0, 0]], rows0, gsem0)

        def pair(p, carry):
            c0 = 2 * p
            pltpu.make_async_copy(y_hbm.at[slots.at[0, 0]], rows0,
                                  gsem0).wait()
            pltpu.async_copy(y_hbm.at[slots.at[1, 0]], rows1, gsem1)
            pltpu.sync_copy(rows0, acc.at[slots.at[0, 1]], add=True)

            def _pre0():
                pltpu.async_copy(e_hbm.at[wid, c0 + 2], slots.at[0], isem0)
            pl.when(p < NPAIR - 1)(_pre0)

            def _g2():
                pltpu.make_async_copy(e_hbm.at[wid, c0 + 2], slots.at[0],
                                      isem0).wait()
                pltpu.async_copy(y_hbm.at[slots.at[0, 0]], rows0, gsem0)
            pl.when(p < NPAIR - 1)(_g2)

            pltpu.make_async_copy(y_hbm.at[slots.at[1, 0]], rows1,
                                  gsem1).wait()
            pltpu.sync_copy(rows1, acc.at[slots.at[1, 1]], add=True)

            def _pre1():
                pltpu.async_copy(e_hbm.at[wid, c0 + 3], slots.at[1], isem1)
                pltpu.make_async_copy(e_hbm.at[wid, c0 + 3], slots.at[1],
                                      isem1).wait()
            pl.when(p < NPAIR - 1)(_pre1)
            return carry

        lax.fori_loop(0, NPAIR, pair, 0)

        plsc.subcore_barrier()
        pltpu.sync_copy(acc.at[pl.ds(s_ax * RT, RT), :],
                        out_hbm.at[c_ax, pl.ds(s_ax * RT, RT), :])

    return agg_kernel


def _prep_edges(src, dst, N, K, NCHUNK):
    E = src.shape[0]
    pad = _NW * NCHUNK * K - E
    src_p = jnp.concatenate([src, jnp.zeros((pad,), src.dtype)])
    dst_p = jnp.concatenate([dst, jnp.full((pad,), N, dst.dtype)])
    return jnp.concatenate(
        [src_p.reshape(_NW, NCHUNK, 1, K), dst_p.reshape(_NW, NCHUNK, 1, K)],
        axis=2)


def _tc_first(dis, x, W):
    N, F = x.shape
    H = W.shape[1]

    def body(dis_ref, x_ref, w_ref, y_ref):
        xw = jnp.dot(x_ref[...], w_ref[...],
                     preferred_element_type=jnp.float32)
        y_ref[...] = dis_ref[...] * xw

    return pl.pallas_call(
        body,
        out_shape=jax.ShapeDtypeStruct((N, H), jnp.float32),
    )(dis, x, W)


def _tc_mid(sp, y, dis, b, W):
    N, H = y.shape

    def body(sp_ref, y_ref, dis_ref, b_ref, w_ref, o_ref):
        sagg = sp_ref[0][:N] + sp_ref[1][:N] + y_ref[...]
        h = jnp.maximum(dis_ref[...] * sagg + b_ref[...], 0.0)
        hw = jnp.dot(h, w_ref[...],
                     preferred_element_type=jnp.float32)
        o_ref[...] = dis_ref[...] * hw

    return pl.pallas_call(
        body,
        out_shape=jax.ShapeDtypeStruct((N, W.shape[1]), jnp.float32),
    )(sp, y, dis, b, W)


def _tc_final(sp, y, dis, b, batch2d, Wh1, bh1, Wh2, bh2):
    N, H = y.shape

    def body(sp_ref, y_ref, dis_ref, b_ref, bt_ref,
             w1_ref, b1_ref, w2_ref, b2_ref, o_ref):
        h2 = jnp.maximum(
            dis_ref[...] * (sp_ref[0][:N] + sp_ref[1][:N] + y_ref[...])
            + b_ref[...],
            0.0)
        gids = lax.broadcasted_iota(jnp.int32, (_G, 1), 0)
        oh = (gids == bt_ref[...]).astype(jnp.float32)
        cnt = jnp.sum(oh, axis=1, keepdims=True)
        summ = jnp.dot(oh, h2,
                       preferred_element_type=jnp.float32,
                       precision=lax.Precision.HIGHEST)
        pooled = summ / jnp.maximum(cnt, 1.0)
        t = jnp.maximum(
            jnp.dot(pooled, w1_ref[...],
                    preferred_element_type=jnp.float32) + b1_ref[...],
            0.0)
        o_ref[...] = jnp.dot(t, w2_ref[...],
                             preferred_element_type=jnp.float32) + b2_ref[...]

    return pl.pallas_call(
        body,
        out_shape=jax.ShapeDtypeStruct((_G, 1), jnp.float32),
    )(sp, y, dis, b, batch2d, Wh1, bh1, Wh2, bh2)


def kernel(x, edge_index, batch, W1, b1, W2, b2, Wh1, bh1, Wh2, bh2):
    N, F = x.shape
    E = edge_index.shape[1]
    K = 128
    NCHUNK = -(-E // (_NW * K * 2)) * 2

    src = edge_index[0]
    dst = edge_index[1]

    dis_mat = _tc_deg(dst.reshape(1, E), dst.reshape(E, 1), N, 6400)
    dis = dis_mat.reshape(-1, 1)[:N]
    y1 = _tc_first(dis, x.astype(jnp.float32), W1)
    agg_a = _make_agg_serial(N, F, K, NCHUNK)
    agg_b = _make_agg_pair(N, F, K, NCHUNK)
    e_il = _prep_edges(src, dst, N, K, NCHUNK)
    z = jnp.zeros((_pad_rows(N), F), jnp.float32)
    s1p = agg_a(y1, e_il, z)
    y2 = _tc_mid(s1p, y1, dis, b1, W2)
    s2p = agg_b(y2, e_il, z)
    out = _tc_final(s2p, y2, dis, b2,
                    batch.reshape(1, N).astype(jnp.int32),
                    Wh1, bh1, Wh2, bh2)
    return out


# speedup vs baseline: 1.5522x; 1.4550x over previous
import functools

import jax
import jax.numpy as jnp
from jax import lax
from jax.experimental import pallas as pl
from jax.experimental.pallas import tpu as pltpu
from jax.experimental.pallas import tpu_sc as plsc

_NC = 2
_NS = 16
_NW = _NC * _NS
_L = 16
_G = 64


def _mesh():
    return plsc.VectorSubcoreMesh(core_axis_name="c", subcore_axis_name="s",
                                  num_cores=_NC, num_subcores=_NS)


def _pad_rows(N):
    return -(-N // (8 * _NS)) * (8 * _NS)


def _tc_deg(dst_row, dst_col, N, EB):
    E = dst_row.shape[1]
    HI = -(-N // 128)
    grid = E // EB

    def body(dr_ref, dc_ref, o_ref):
        i = pl.program_id(0)

        @pl.when(i == 0)
        def _init():
            o_ref[...] = jnp.zeros_like(o_ref)

        hi = dr_ref[...] // 128
        lo = dc_ref[...] % 128
        oh_hi = (lax.broadcasted_iota(jnp.int32, (HI, 1), 0) == hi
                 ).astype(jnp.float32)
        oh_lo = (lo == lax.broadcasted_iota(jnp.int32, (1, 128), 1)
                 ).astype(jnp.float32)
        o_ref[...] += jnp.dot(oh_hi, oh_lo,
                              preferred_element_type=jnp.float32)

        @pl.when(i == grid - 1)
        def _finish():
            o_ref[...] = lax.rsqrt(o_ref[...] + 1.0)

    return pl.pallas_call(
        body,
        grid=(grid,),
        in_specs=[pl.BlockSpec((1, EB), lambda i: (0, i)),
                  pl.BlockSpec((EB, 1), lambda i: (i, 0))],
        out_specs=pl.BlockSpec((HI, 128), lambda i: (0, 0)),
        out_shape=jax.ShapeDtypeStruct((HI, 128), jnp.float32),
    )(dst_row, dst_col)


@functools.lru_cache(maxsize=None)
def _make_agg_kernel(N, F, K, NCHUNK):
    NP = _pad_rows(N)
    RT = NP // _NS
    NB, NQ = 4, 8
    NG = NCHUNK // NQ
    assert NCHUNK % NQ == 0

    @functools.partial(
        pl.kernel,
        out_type=jax.ShapeDtypeStruct((_NC, NP, F), jnp.float32),
        mesh=_mesh(),
        scratch_types=(
            [pltpu.VMEM_SHARED((NP, F), jnp.float32)]
            + [pltpu.VMEM((K, F), jnp.float32) for _ in range(NB)]
            + [pltpu.VMEM((NQ, 2, K), jnp.int32)]
            + [pltpu.SemaphoreType.DMA] * (2 * NB + NQ)
        ),
    )
    def agg_kernel(y_hbm, e_hbm, z_hbm, out_hbm, acc, *rest):
        rows = rest[:NB]
        slots = rest[NB]
        gsem = rest[NB + 1: NB + 1 + NB]
        ssem = rest[NB + 1 + NB: NB + 1 + 2 * NB]
        isem = rest[NB + 1 + 2 * NB:]
        c_ax = lax.axis_index("c")
        s_ax = lax.axis_index("s")
        wid = s_ax * _NC + c_ax

        pltpu.sync_copy(z_hbm.at[pl.ds(s_ax * RT, RT), :],
                        acc.at[pl.ds(s_ax * RT, RT), :])
        plsc.subcore_barrier()

        def idx_load(ci, q):
            return pltpu.async_copy(e_hbm.at[wid, ci], slots.at[q], isem[q])

        def gather(q, b):
            return pltpu.async_copy(y_hbm.at[slots.at[q, 0]], rows[b], gsem[b])

        def scatter(q, b):
            return pltpu.async_copy(rows[b], acc.at[slots.at[q, 1]], ssem[b],
                                    add=True)

        for q in range(NQ):
            idx_load(q, q)
        pltpu.make_async_copy(e_hbm.at[wid, 0], slots.at[0], isem[0]).wait()
        gather(0, 0)
        pltpu.make_async_copy(e_hbm.at[wid, 1], slots.at[1], isem[1]).wait()
        gather(1, 1)

        def group(g, carry):
            for k in range(NQ):
                b, q = k % NB, k
                b2, q2 = (k - 2) % NB, (k - 2) % NQ
                pltpu.make_async_copy(y_hbm.at[slots.at[q, 0]], rows[b],
                                      gsem[b]).wait()
                scatter(q, b)

                def _wait_sc():
                    pltpu.make_async_copy(rows[b2], acc.at[slots.at[q2, 1]],
                                          ssem[b2]).wait()
                if k >= 2:
                    _wait_sc()
                else:
                    pl.when(g > 0)(_wait_sc)

                def _iload():
                    idx_load(g * NQ + k + 6, q2)
                if k >= 2:
                    pl.when(g * NQ + k + 6 < NCHUNK)(_iload)
                else:
                    pl.when(jnp.logical_and(
                        g > 0, g * NQ + k + 6 < NCHUNK))(_iload)

                q3 = (k + 2) % NQ
                def _gath():
                    pltpu.make_async_copy(e_hbm.at[wid, g * NQ + k + 2],
                                          slots.at[q3], isem[q3]).wait()
                    gather(q3, b2)
                if k < NQ - 2:
                    _gath()
                else:
                    pl.when(g < NG - 1)(_gath)
            return carry

        lax.fori_loop(0, NG, group, 0)

        for k in (NQ - 2, NQ - 1):
            pltpu.make_async_copy(rows[k % NB], acc.at[slots.at[k, 1]],
                                  ssem[k % NB]).wait()

        plsc.subcore_barrier()
        pltpu.sync_copy(acc.at[pl.ds(s_ax * RT, RT), :],
                        out_hbm.at[c_ax, pl.ds(s_ax * RT, RT), :])

    return agg_kernel


@functools.lru_cache(maxsize=None)
def _make_agg_serial(N, F, K, NCHUNK):
    NP = _pad_rows(N)
    RT = NP // _NS

    @functools.partial(
        pl.kernel,
        out_type=jax.ShapeDtypeStruct((_NC, NP, F), jnp.float32),
        mesh=_mesh(),
        scratch_types=[
            pltpu.VMEM_SHARED((NP, F), jnp.float32),
            pltpu.VMEM((2, K), jnp.int32),
            pltpu.VMEM((K, F), jnp.float32),
            pltpu.SemaphoreType.DMA,
        ],
    )
    def agg_kernel(y_hbm, e_hbm, z_hbm, out_hbm, acc, slot, rows, sem):
        c_ax = lax.axis_index("c")
        s_ax = lax.axis_index("s")
        wid = s_ax * _NC + c_ax

        pltpu.sync_copy(z_hbm.at[pl.ds(s_ax * RT, RT), :],
                        acc.at[pl.ds(s_ax * RT, RT), :])
        plsc.subcore_barrier()

        def chunk(ci, carry):
            pltpu.sync_copy(e_hbm.at[wid, ci], slot)
            pltpu.async_copy(y_hbm.at[slot.at[0]], rows, sem).wait()
            pltpu.sync_copy(rows, acc.at[slot.at[1]], add=True)
            return carry
        lax.fori_loop(0, NCHUNK, chunk, 0)

        plsc.subcore_barrier()
        pltpu.sync_copy(acc.at[pl.ds(s_ax * RT, RT), :],
                        out_hbm.at[c_ax, pl.ds(s_ax * RT, RT), :])

    return agg_kernel


@functools.lru_cache(maxsize=None)
def _make_agg_pair(N, F, K, NCHUNK):
    NP = _pad_rows(N)
    RT = NP // _NS
    NPAIR = NCHUNK // 2
    assert NCHUNK % 2 == 0

    @functools.partial(
        pl.kernel,
        out_type=jax.ShapeDtypeStruct((_NC, NP, F), jnp.float32),
        mesh=_mesh(),
        scratch_types=[
            pltpu.VMEM_SHARED((NP, F), jnp.float32),
            pltpu.VMEM((2, 2, K), jnp.int32),
            pltpu.VMEM((K, F), jnp.float32),
            pltpu.VMEM((K, F), jnp.float32),
            pltpu.SemaphoreType.DMA,
            pltpu.SemaphoreType.DMA,
            pltpu.SemaphoreType.DMA,
            pltpu.SemaphoreType.DMA,
        ],
    )
    def agg_kernel(y_hbm, e_hbm, z_hbm, out_hbm,
                   acc, slots, rows0, rows1, gsem0, gsem1, isem0, isem1):
        c_ax = lax.axis_index("c")
        s_ax = lax.axis_index("s")
        wid = s_ax * _NC + c_ax

        pltpu.sync_copy(z_hbm.at[pl.ds(s_ax * RT, RT), :],
                        acc.at[pl.ds(s_ax * RT, RT), :])
        plsc.subcore_barrier()

        pltpu.sync_copy(e_hbm.at[wid, 0], slots.at[0])
        pltpu.sync_copy(e_hbm.at[wid, 1], slots.at[1])
        pltpu.async_copy(y_hbm.at[slots.at[0, 0]], rows0, gsem0)

        def pair(p, carry):
            c0 = 2 * p
            pltpu.make_async_copy(y_hbm.at[slots.at[0, 0]], rows0,
                                  gsem0).wait()
            pltpu.async_copy(y_hbm.at[slots.at[1, 0]], rows1, gsem1)
            pltpu.sync_copy(rows0, acc.at[slots.at[0, 1]], add=True)

            def _pre0():
                pltpu.async_copy(e_hbm.at[wid, c0 + 2], slots.at[0], isem0)
            pl.when(p < NPAIR - 1)(_pre0)

            def _g2():
                pltpu.make_async_copy(e_hbm.at[wid, c0 + 2], slots.at[0],
                                      isem0).wait()
                pltpu.async_copy(y_hbm.at[slots.at[0, 0]], rows0, gsem0)
            pl.when(p < NPAIR - 1)(_g2)

            pltpu.make_async_copy(y_hbm.at[slots.at[1, 0]], rows1,
                                  gsem1).wait()
            pltpu.sync_copy(rows1, acc.at[slots.at[1, 1]], add=True)

            def _pre1():
                pltpu.async_copy(e_hbm.at[wid, c0 + 3], slots.at[1], isem1)
                pltpu.make_async_copy(e_hbm.at[wid, c0 + 3], slots.at[1],
                                      isem1).wait()
            pl.when(p < NPAIR - 1)(_pre1)
            return carry

        lax.fori_loop(0, NPAIR, pair, 0)

        plsc.subcore_barrier()
        pltpu.sync_copy(acc.at[pl.ds(s_ax * RT, RT), :],
                        out_hbm.at[c_ax, pl.ds(s_ax * RT, RT), :])

    return agg_kernel


def _prep_edges(src, dst, N, K, NCHUNK):
    E = src.shape[0]
    pad = _NW * NCHUNK * K - E
    src_p = jnp.concatenate([src, jnp.zeros((pad,), src.dtype)])
    dst_p = jnp.concatenate([dst, jnp.full((pad,), N, dst.dtype)])
    return jnp.concatenate(
        [src_p.reshape(_NW, NCHUNK, 1, K), dst_p.reshape(_NW, NCHUNK, 1, K)],
        axis=2)


def _tc_first(dis, x, W):
    N, F = x.shape
    H = W.shape[1]

    def body(dis_ref, x_ref, w_ref, y_ref):
        xw = jnp.dot(x_ref[...], w_ref[...],
                     preferred_element_type=jnp.float32)
        y_ref[...] = dis_ref[...] * xw

    return pl.pallas_call(
        body,
        out_shape=jax.ShapeDtypeStruct((N, H), jnp.float32),
    )(dis, x, W)


def _tc_mid(sp, y, dis, b, W):
    N, H = y.shape

    def body(sp_ref, y_ref, dis_ref, b_ref, w_ref, o_ref):
        sagg = sp_ref[0][:N] + sp_ref[1][:N] + y_ref[...]
        h = jnp.maximum(dis_ref[...] * sagg + b_ref[...], 0.0)
        hw = jnp.dot(h, w_ref[...],
                     preferred_element_type=jnp.float32)
        o_ref[...] = dis_ref[...] * hw

    return pl.pallas_call(
        body,
        out_shape=jax.ShapeDtypeStruct((N, W.shape[1]), jnp.float32),
    )(sp, y, dis, b, W)


def _tc_final(sp, y, dis, b, batch2d, Wh1, bh1, Wh2, bh2):
    N, H = y.shape

    def body(sp_ref, y_ref, dis_ref, b_ref, bt_ref,
             w1_ref, b1_ref, w2_ref, b2_ref, o_ref):
        h2 = jnp.maximum(
            dis_ref[...] * (sp_ref[0][:N] + sp_ref[1][:N] + y_ref[...])
            + b_ref[...],
            0.0)
        gids = lax.broadcasted_iota(jnp.int32, (_G, 1), 0)
        oh = (gids == bt_ref[...]).astype(jnp.float32)
        cnt = jnp.sum(oh, axis=1, keepdims=True)
        summ = jnp.dot(oh, h2,
                       preferred_element_type=jnp.float32,
                       precision=lax.Precision.HIGHEST)
        pooled = summ / jnp.maximum(cnt, 1.0)
        t = jnp.maximum(
            jnp.dot(pooled, w1_ref[...],
                    preferred_element_type=jnp.float32) + b1_ref[...],
            0.0)
        o_ref[...] = jnp.dot(t, w2_ref[...],
                             preferred_element_type=jnp.float32) + b2_ref[...]

    return pl.pallas_call(
        body,
        out_shape=jax.ShapeDtypeStruct((_G, 1), jnp.float32),
    )(sp, y, dis, b, batch2d, Wh1, bh1, Wh2, bh2)


def kernel(x, edge_index, batch, W1, b1, W2, b2, Wh1, bh1, Wh2, bh2):
    N, F = x.shape
    E = edge_index.shape[1]
    K = 80
    NCHUNK = -(-E // (_NW * K * 2)) * 2

    src = edge_index[0]
    dst = edge_index[1]

    dis_mat = _tc_deg(dst.reshape(1, E), dst.reshape(E, 1), N, 6400)
    dis = dis_mat.reshape(-1, 1)[:N]
    y1 = _tc_first(dis, x.astype(jnp.float32), W1)
    agg_a = _make_agg_serial(N, F, K, NCHUNK)
    agg_b = _make_agg_pair(N, F, K, NCHUNK)
    e_il = _prep_edges(src, dst, N, K, NCHUNK)
    z = jnp.zeros((_pad_rows(N), F), jnp.float32)
    s1p = agg_a(y1, e_il, z)
    y2 = _tc_mid(s1p, y1, dis, b1, W2)
    s2p = agg_b(y2, e_il, z)
    out = _tc_final(s2p, y2, dis, b2,
                    batch.reshape(1, N).astype(jnp.int32),
                    Wh1, bh1, Wh2, bh2)
    return out


# speedup vs baseline: 1.7831x; 1.1488x over previous
import functools

import jax
import jax.numpy as jnp
from jax import lax
from jax.experimental import pallas as pl
from jax.experimental.pallas import tpu as pltpu
from jax.experimental.pallas import tpu_sc as plsc

_NC = 2
_NS = 16
_NW = _NC * _NS
_L = 16
_G = 64


def _mesh():
    return plsc.VectorSubcoreMesh(core_axis_name="c", subcore_axis_name="s",
                                  num_cores=_NC, num_subcores=_NS)


def _pad_rows(N):
    return -(-N // (8 * _NS)) * (8 * _NS)


def _tc_deg(dst_row, dst_col, N, EB):
    E = dst_row.shape[1]
    HI = -(-N // 128)
    grid = E // EB

    def body(dr_ref, dc_ref, o_ref):
        i = pl.program_id(0)

        @pl.when(i == 0)
        def _init():
            o_ref[...] = jnp.zeros_like(o_ref)

        hi = dr_ref[...] // 128
        lo = dc_ref[...] % 128
        oh_hi = (lax.broadcasted_iota(jnp.int32, (HI, 1), 0) == hi
                 ).astype(jnp.float32)
        oh_lo = (lo == lax.broadcasted_iota(jnp.int32, (1, 128), 1)
                 ).astype(jnp.float32)
        o_ref[...] += jnp.dot(oh_hi, oh_lo,
                              preferred_element_type=jnp.float32)

        @pl.when(i == grid - 1)
        def _finish():
            o_ref[...] = lax.rsqrt(o_ref[...] + 1.0)

    return pl.pallas_call(
        body,
        grid=(grid,),
        in_specs=[pl.BlockSpec((1, EB), lambda i: (0, i)),
                  pl.BlockSpec((EB, 1), lambda i: (i, 0))],
        out_specs=pl.BlockSpec((HI, 128), lambda i: (0, 0)),
        out_shape=jax.ShapeDtypeStruct((HI, 128), jnp.float32),
    )(dst_row, dst_col)


@functools.lru_cache(maxsize=None)
def _make_agg_kernel(N, F, K, NCHUNK):
    NP = _pad_rows(N)
    RT = NP // _NS
    NB, NQ = 4, 8
    NG = NCHUNK // NQ
    assert NCHUNK % NQ == 0

    @functools.partial(
        pl.kernel,
        out_type=jax.ShapeDtypeStruct((_NC, NP, F), jnp.float32),
        mesh=_mesh(),
        scratch_types=(
            [pltpu.VMEM_SHARED((NP, F), jnp.float32)]
            + [pltpu.VMEM((K, F), jnp.float32) for _ in range(NB)]
            + [pltpu.VMEM((NQ, 2, K), jnp.int32)]
            + [pltpu.SemaphoreType.DMA] * (2 * NB + NQ)
        ),
    )
    def agg_kernel(y_hbm, e_hbm, z_hbm, out_hbm, acc, *rest):
        rows = rest[:NB]
        slots = rest[NB]
        gsem = rest[NB + 1: NB + 1 + NB]
        ssem = rest[NB + 1 + NB: NB + 1 + 2 * NB]
        isem = rest[NB + 1 + 2 * NB:]
        c_ax = lax.axis_index("c")
        s_ax = lax.axis_index("s")
        wid = s_ax * _NC + c_ax

        pltpu.sync_copy(z_hbm.at[pl.ds(s_ax * RT, RT), :],
                        acc.at[pl.ds(s_ax * RT, RT), :])
        plsc.subcore_barrier()

        def idx_load(ci, q):
            return pltpu.async_copy(e_hbm.at[wid, ci], slots.at[q], isem[q])

        def gather(q, b):
            return pltpu.async_copy(y_hbm.at[slots.at[q, 0]], rows[b], gsem[b])

        def scatter(q, b):
            return pltpu.async_copy(rows[b], acc.at[slots.at[q, 1]], ssem[b],
                                    add=True)

        for q in range(NQ):
            idx_load(q, q)
        pltpu.make_async_copy(e_hbm.at[wid, 0], slots.at[0], isem[0]).wait()
        gather(0, 0)
        pltpu.make_async_copy(e_hbm.at[wid, 1], slots.at[1], isem[1]).wait()
        gather(1, 1)

        def group(g, carry):
            for k in range(NQ):
                b, q = k % NB, k
                b2, q2 = (k - 2) % NB, (k - 2) % NQ
                pltpu.make_async_copy(y_hbm.at[slots.at[q, 0]], rows[b],
                                      gsem[b]).wait()
                scatter(q, b)

                def _wait_sc():
                    pltpu.make_async_copy(rows[b2], acc.at[slots.at[q2, 1]],
                                          ssem[b2]).wait()
                if k >= 2:
                    _wait_sc()
                else:
                    pl.when(g > 0)(_wait_sc)

                def _iload():
                    idx_load(g * NQ + k + 6, q2)
                if k >= 2:
                    pl.when(g * NQ + k + 6 < NCHUNK)(_iload)
                else:
                    pl.when(jnp.logical_and(
                        g > 0, g * NQ + k + 6 < NCHUNK))(_iload)

                q3 = (k + 2) % NQ
                def _gath():
                    pltpu.make_async_copy(e_hbm.at[wid, g * NQ + k + 2],
                                          slots.at[q3], isem[q3]).wait()
                    gather(q3, b2)
                if k < NQ - 2:
                    _gath()
                else:
                    pl.when(g < NG - 1)(_gath)
            return carry

        lax.fori_loop(0, NG, group, 0)

        for k in (NQ - 2, NQ - 1):
            pltpu.make_async_copy(rows[k % NB], acc.at[slots.at[k, 1]],
                                  ssem[k % NB]).wait()

        plsc.subcore_barrier()
        pltpu.sync_copy(acc.at[pl.ds(s_ax * RT, RT), :],
                        out_hbm.at[c_ax, pl.ds(s_ax * RT, RT), :])

    return agg_kernel


@functools.lru_cache(maxsize=None)
def _make_agg_serial(N, F, K, NCHUNK):
    NP = _pad_rows(N)
    RT = NP // _NS

    @functools.partial(
        pl.kernel,
        out_type=jax.ShapeDtypeStruct((_NC, NP, F), jnp.float32),
        mesh=_mesh(),
        scratch_types=[
            pltpu.VMEM_SHARED((NP, F), jnp.float32),
            pltpu.VMEM((2, K), jnp.int32),
            pltpu.VMEM((K, F), jnp.float32),
            pltpu.SemaphoreType.DMA,
        ],
    )
    def agg_kernel(y_hbm, e_hbm, z_hbm, out_hbm, acc, slot, rows, sem):
        c_ax = lax.axis_index("c")
        s_ax = lax.axis_index("s")
        wid = s_ax * _NC + c_ax

        pltpu.sync_copy(z_hbm.at[pl.ds(s_ax * RT, RT), :],
                        acc.at[pl.ds(s_ax * RT, RT), :])
        plsc.subcore_barrier()

        def chunk(ci, carry):
            pltpu.sync_copy(e_hbm.at[wid, ci], slot)
            pltpu.async_copy(y_hbm.at[slot.at[0]], rows, sem).wait()
            pltpu.sync_copy(rows, acc.at[slot.at[1]], add=True)
            return carry
        lax.fori_loop(0, NCHUNK, chunk, 0)

        plsc.subcore_barrier()
        pltpu.sync_copy(acc.at[pl.ds(s_ax * RT, RT), :],
                        out_hbm.at[c_ax, pl.ds(s_ax * RT, RT), :])

    return agg_kernel


@functools.lru_cache(maxsize=None)
def _make_agg_pair(N, F, K, NCHUNK):
    NP = _pad_rows(N)
    RT = NP // _NS
    NPAIR = NCHUNK // 2
    assert NCHUNK % 2 == 0

    @functools.partial(
        pl.kernel,
        out_type=jax.ShapeDtypeStruct((_NC, NP, F), jnp.float32),
        mesh=_mesh(),
        scratch_types=[
            pltpu.VMEM_SHARED((NP, F), jnp.float32),
            pltpu.VMEM((2, 2, K), jnp.int32),
            pltpu.VMEM((K, F), jnp.float32),
            pltpu.VMEM((K, F), jnp.float32),
            pltpu.SemaphoreType.DMA,
            pltpu.SemaphoreType.DMA,
            pltpu.SemaphoreType.DMA,
            pltpu.SemaphoreType.DMA,
        ],
    )
    def agg_kernel(y_hbm, e_hbm, z_hbm, out_hbm,
                   acc, slots, rows0, rows1, gsem0, gsem1, isem0, isem1):
        c_ax = lax.axis_index("c")
        s_ax = lax.axis_index("s")
        wid = s_ax * _NC + c_ax

        pltpu.sync_copy(z_hbm.at[pl.ds(s_ax * RT, RT), :],
                        acc.at[pl.ds(s_ax * RT, RT), :])
        plsc.subcore_barrier()

        pltpu.sync_copy(e_hbm.at[wid, 0], slots.at[0])
        pltpu.sync_copy(e_hbm.at[wid, 1], slots.at[1])
        pltpu.async_copy(y_hbm.at[slots.at[0, 0]], rows0, gsem0)

        def pair(p, carry):
            c0 = 2 * p
            pltpu.make_async_copy(y_hbm.at[slots.at[0, 0]], rows0,
                                  gsem0).wait()
            pltpu.async_copy(y_hbm.at[slots.at[1, 0]], rows1, gsem1)
            pltpu.sync_copy(rows0, acc.at[slots.at[0, 1]], add=True)

            def _pre0():
                pltpu.async_copy(e_hbm.at[wid, c0 + 2], slots.at[0], isem0)
            pl.when(p < NPAIR - 1)(_pre0)

            def _g2():
                pltpu.make_async_copy(e_hbm.at[wid, c0 + 2], slots.at[0],
                                      isem0).wait()
                pltpu.async_copy(y_hbm.at[slots.at[0, 0]], rows0, gsem0)
            pl.when(p < NPAIR - 1)(_g2)

            pltpu.make_async_copy(y_hbm.at[slots.at[1, 0]], rows1,
                                  gsem1).wait()
            pltpu.sync_copy(rows1, acc.at[slots.at[1, 1]], add=True)

            def _pre1():
                pltpu.async_copy(e_hbm.at[wid, c0 + 3], slots.at[1], isem1)
                pltpu.make_async_copy(e_hbm.at[wid, c0 + 3], slots.at[1],
                                      isem1).wait()
            pl.when(p < NPAIR - 1)(_pre1)
            return carry

        lax.fori_loop(0, NPAIR, pair, 0)

        plsc.subcore_barrier()
        pltpu.sync_copy(acc.at[pl.ds(s_ax * RT, RT), :],
                        out_hbm.at[c_ax, pl.ds(s_ax * RT, RT), :])

    return agg_kernel


def _prep_edges(src, dst, N, K, NCHUNK):
    E = src.shape[0]
    pad = _NW * NCHUNK * K - E
    src_p = jnp.concatenate([src, jnp.zeros((pad,), src.dtype)])
    dst_p = jnp.concatenate([dst, jnp.full((pad,), N, dst.dtype)])
    return jnp.concatenate(
        [src_p.reshape(_NW, NCHUNK, 1, K), dst_p.reshape(_NW, NCHUNK, 1, K)],
        axis=2)


def _tc_first(dis, x, W):
    N, F = x.shape
    H = W.shape[1]

    def body(dis_ref, x_ref, w_ref, y_ref):
        xw = jnp.dot(x_ref[...], w_ref[...],
                     preferred_element_type=jnp.float32)
        y_ref[...] = dis_ref[...] * xw

    return pl.pallas_call(
        body,
        out_shape=jax.ShapeDtypeStruct((N, H), jnp.float32),
    )(dis, x, W)


def _tc_mid(sp, y, dis, b, W):
    N, H = y.shape

    def body(sp_ref, y_ref, dis_ref, b_ref, w_ref, o_ref):
        sagg = sp_ref[0][:N] + sp_ref[1][:N] + y_ref[...]
        h = jnp.maximum(dis_ref[...] * sagg + b_ref[...], 0.0)
        hw = jnp.dot(h, w_ref[...],
                     preferred_element_type=jnp.float32)
        o_ref[...] = dis_ref[...] * hw

    return pl.pallas_call(
        body,
        out_shape=jax.ShapeDtypeStruct((N, W.shape[1]), jnp.float32),
    )(sp, y, dis, b, W)


def _tc_final(sp, y, dis, b, batch2d, Wh1, bh1, Wh2, bh2):
    N, H = y.shape

    def body(sp_ref, y_ref, dis_ref, b_ref, bt_ref,
             w1_ref, b1_ref, w2_ref, b2_ref, o_ref):
        h2 = jnp.maximum(
            dis_ref[...] * (sp_ref[0][:N] + sp_ref[1][:N] + y_ref[...])
            + b_ref[...],
            0.0)
        gids = lax.broadcasted_iota(jnp.int32, (_G, 1), 0)
        oh = (gids == bt_ref[...]).astype(jnp.float32)
        cnt = jnp.sum(oh, axis=1, keepdims=True)
        summ = jnp.dot(oh, h2,
                       preferred_element_type=jnp.float32,
                       precision=lax.Precision.HIGHEST)
        pooled = summ / jnp.maximum(cnt, 1.0)
        t = jnp.maximum(
            jnp.dot(pooled, w1_ref[...],
                    preferred_element_type=jnp.float32) + b1_ref[...],
            0.0)
        o_ref[...] = jnp.dot(t, w2_ref[...],
                             preferred_element_type=jnp.float32) + b2_ref[...]

    return pl.pallas_call(
        body,
        out_shape=jax.ShapeDtypeStruct((_G, 1), jnp.float32),
    )(sp, y, dis, b, batch2d, Wh1, bh1, Wh2, bh2)


def kernel(x, edge_index, batch, W1, b1, W2, b2, Wh1, bh1, Wh2, bh2):
    N, F = x.shape
    E = edge_index.shape[1]
    K = 80
    NCHUNK = -(-E // (_NW * K * 2)) * 2

    src = edge_index[0]
    dst = edge_index[1]

    dis_mat = _tc_deg(dst.reshape(1, E), dst.reshape(E, 1), N, 6400)
    dis = dis_mat.reshape(-1, 1)[:N]
    y1 = _tc_first(dis, x.astype(jnp.float32), W1)
    agg_b = _make_agg_pair(N, F, K, NCHUNK)
    e_il = _prep_edges(src, dst, N, K, NCHUNK)
    z = jnp.zeros((_pad_rows(N), F), jnp.float32)
    s1p = agg_b(y1, e_il, z)
    y2 = _tc_mid(s1p, y1, dis, b1, W2)
    s2p = agg_b(y2, e_il, z)
    out = _tc_final(s2p, y2, dis, b2,
                    batch.reshape(1, N).astype(jnp.int32),
                    Wh1, bh1, Wh2, bh2)
    return out


# speedup vs baseline: 2.1274x; 1.1931x over previous
import functools

import jax
import jax.numpy as jnp
from jax import lax
from jax.experimental import pallas as pl
from jax.experimental.pallas import tpu as pltpu
from jax.experimental.pallas import tpu_sc as plsc

_NC = 2
_NS = 16
_NW = _NC * _NS
_L = 16
_G = 64


def _mesh():
    return plsc.VectorSubcoreMesh(core_axis_name="c", subcore_axis_name="s",
                                  num_cores=_NC, num_subcores=_NS)


def _pad_rows(N):
    return -(-N // (8 * _NS)) * (8 * _NS)


def _tc_deg(dst_row, dst_col, N, EB):
    E = dst_row.shape[1]
    HI = -(-N // 128)
    grid = E // EB

    def body(dr_ref, dc_ref, o_ref):
        i = pl.program_id(0)

        @pl.when(i == 0)
        def _init():
            o_ref[...] = jnp.zeros_like(o_ref)

        hi = dr_ref[...] // 128
        lo = dc_ref[...] % 128
        oh_hi = (lax.broadcasted_iota(jnp.int32, (HI, 1), 0) == hi
                 ).astype(jnp.float32)
        oh_lo = (lo == lax.broadcasted_iota(jnp.int32, (1, 128), 1)
                 ).astype(jnp.float32)
        o_ref[...] += jnp.dot(oh_hi, oh_lo,
                              preferred_element_type=jnp.float32)

        @pl.when(i == grid - 1)
        def _finish():
            o_ref[...] = lax.rsqrt(o_ref[...] + 1.0)

    return pl.pallas_call(
        body,
        grid=(grid,),
        in_specs=[pl.BlockSpec((1, EB), lambda i: (0, i)),
                  pl.BlockSpec((EB, 1), lambda i: (i, 0))],
        out_specs=pl.BlockSpec((HI, 128), lambda i: (0, 0)),
        out_shape=jax.ShapeDtypeStruct((HI, 128), jnp.float32),
    )(dst_row, dst_col)


@functools.lru_cache(maxsize=None)
def _make_agg_kernel(N, F, K, NCHUNK):
    NP = _pad_rows(N)
    RT = NP // _NS
    NB, NQ = 4, 8
    NG = NCHUNK // NQ
    assert NCHUNK % NQ == 0

    @functools.partial(
        pl.kernel,
        out_type=jax.ShapeDtypeStruct((_NC, NP, F), jnp.float32),
        mesh=_mesh(),
        scratch_types=(
            [pltpu.VMEM_SHARED((NP, F), jnp.float32)]
            + [pltpu.VMEM((K, F), jnp.float32) for _ in range(NB)]
            + [pltpu.VMEM((NQ, 2, K), jnp.int32)]
            + [pltpu.SemaphoreType.DMA] * (2 * NB + NQ)
        ),
    )
    def agg_kernel(y_hbm, e_hbm, z_hbm, out_hbm, acc, *rest):
        rows = rest[:NB]
        slots = rest[NB]
        gsem = rest[NB + 1: NB + 1 + NB]
        ssem = rest[NB + 1 + NB: NB + 1 + 2 * NB]
        isem = rest[NB + 1 + 2 * NB:]
        c_ax = lax.axis_index("c")
        s_ax = lax.axis_index("s")
        wid = s_ax * _NC + c_ax

        pltpu.sync_copy(z_hbm.at[pl.ds(s_ax * RT, RT), :],
                        acc.at[pl.ds(s_ax * RT, RT), :])
        plsc.subcore_barrier()

        def idx_load(ci, q):
            return pltpu.async_copy(e_hbm.at[wid, ci], slots.at[q], isem[q])

        def gather(q, b):
            return pltpu.async_copy(y_hbm.at[slots.at[q, 0]], rows[b], gsem[b])

        def scatter(q, b):
            return pltpu.async_copy(rows[b], acc.at[slots.at[q, 1]], ssem[b],
                                    add=True)

        for q in range(NQ):
            idx_load(q, q)
        pltpu.make_async_copy(e_hbm.at[wid, 0], slots.at[0], isem[0]).wait()
        gather(0, 0)
        pltpu.make_async_copy(e_hbm.at[wid, 1], slots.at[1], isem[1]).wait()
        gather(1, 1)

        def group(g, carry):
            for k in range(NQ):
                b, q = k % NB, k
                b2, q2 = (k - 2) % NB, (k - 2) % NQ
                pltpu.make_async_copy(y_hbm.at[slots.at[q, 0]], rows[b],
                                      gsem[b]).wait()
                scatter(q, b)

                def _wait_sc():
                    pltpu.make_async_copy(rows[b2], acc.at[slots.at[q2, 1]],
                                          ssem[b2]).wait()
                if k >= 2:
                    _wait_sc()
                else:
                    pl.when(g > 0)(_wait_sc)

                def _iload():
                    idx_load(g * NQ + k + 6, q2)
                if k >= 2:
                    pl.when(g * NQ + k + 6 < NCHUNK)(_iload)
                else:
                    pl.when(jnp.logical_and(
                        g > 0, g * NQ + k + 6 < NCHUNK))(_iload)

                q3 = (k + 2) % NQ
                def _gath():
                    pltpu.make_async_copy(e_hbm.at[wid, g * NQ + k + 2],
                                          slots.at[q3], isem[q3]).wait()
                    gather(q3, b2)
                if k < NQ - 2:
                    _gath()
                else:
                    pl.when(g < NG - 1)(_gath)
            return carry

        lax.fori_loop(0, NG, group, 0)

        for k in (NQ - 2, NQ - 1):
            pltpu.make_async_copy(rows[k % NB], acc.at[slots.at[k, 1]],
                                  ssem[k % NB]).wait()

        plsc.subcore_barrier()
        pltpu.sync_copy(acc.at[pl.ds(s_ax * RT, RT), :],
                        out_hbm.at[c_ax, pl.ds(s_ax * RT, RT), :])

    return agg_kernel


@functools.lru_cache(maxsize=None)
def _make_agg_serial(N, F, K, NCHUNK):
    NP = _pad_rows(N)
    RT = NP // _NS

    @functools.partial(
        pl.kernel,
        out_type=jax.ShapeDtypeStruct((_NC, NP, F), jnp.float32),
        mesh=_mesh(),
        scratch_types=[
            pltpu.VMEM_SHARED((NP, F), jnp.float32),
            pltpu.VMEM((2, K), jnp.int32),
            pltpu.VMEM((K, F), jnp.float32),
            pltpu.SemaphoreType.DMA,
        ],
    )
    def agg_kernel(y_hbm, e_hbm, z_hbm, out_hbm, acc, slot, rows, sem):
        c_ax = lax.axis_index("c")
        s_ax = lax.axis_index("s")
        wid = s_ax * _NC + c_ax

        pltpu.sync_copy(z_hbm.at[pl.ds(s_ax * RT, RT), :],
                        acc.at[pl.ds(s_ax * RT, RT), :])
        plsc.subcore_barrier()

        def chunk(ci, carry):
            pltpu.sync_copy(e_hbm.at[wid, ci], slot)
            pltpu.async_copy(y_hbm.at[slot.at[0]], rows, sem).wait()
            pltpu.sync_copy(rows, acc.at[slot.at[1]], add=True)
            return carry
        lax.fori_loop(0, NCHUNK, chunk, 0)

        plsc.subcore_barrier()
        pltpu.sync_copy(acc.at[pl.ds(s_ax * RT, RT), :],
                        out_hbm.at[c_ax, pl.ds(s_ax * RT, RT), :])

    return agg_kernel


@functools.lru_cache(maxsize=None)
def _make_agg_pair(N, F, K, NCA, NCB):
    NP = _pad_rows(N)
    RT = NP // _NS
    NCHUNK = max(NCA, NCB)
    assert NCA % 2 == 0 and NCB % 2 == 0

    @functools.partial(
        pl.kernel,
        out_type=jax.ShapeDtypeStruct((_NC, NP, F), jnp.float32),
        mesh=_mesh(),
        scratch_types=[
            pltpu.VMEM_SHARED((NP, F), jnp.float32),
            pltpu.VMEM((2, 2, K), jnp.int32),
            pltpu.VMEM((K, F), jnp.float32),
            pltpu.VMEM((K, F), jnp.float32),
            pltpu.SemaphoreType.DMA,
            pltpu.SemaphoreType.DMA,
            pltpu.SemaphoreType.DMA,
            pltpu.SemaphoreType.DMA,
        ],
    )
    def agg_kernel(y_hbm, e_hbm, z_hbm, out_hbm,
                   acc, slots, rows0, rows1, gsem0, gsem1, isem0, isem1):
        c_ax = lax.axis_index("c")
        s_ax = lax.axis_index("s")
        wid = s_ax * _NC + c_ax

        npair = jnp.where(c_ax == 0, NCA // 2, NCB // 2)

        pltpu.sync_copy(z_hbm.at[pl.ds(s_ax * RT, RT), :],
                        acc.at[pl.ds(s_ax * RT, RT), :])
        plsc.subcore_barrier()

        pltpu.sync_copy(e_hbm.at[wid, 0], slots.at[0])
        pltpu.sync_copy(e_hbm.at[wid, 1], slots.at[1])
        pltpu.async_copy(y_hbm.at[slots.at[0, 0]], rows0, gsem0)

        def pair(p, carry):
            c0 = 2 * p
            pltpu.make_async_copy(y_hbm.at[slots.at[0, 0]], rows0,
                                  gsem0).wait()
            pltpu.async_copy(y_hbm.at[slots.at[1, 0]], rows1, gsem1)
            pltpu.sync_copy(rows0, acc.at[slots.at[0, 1]], add=True)

            def _pre0():
                pltpu.async_copy(e_hbm.at[wid, c0 + 2], slots.at[0], isem0)
            pl.when(p < npair - 1)(_pre0)

            def _g2():
                pltpu.make_async_copy(e_hbm.at[wid, c0 + 2], slots.at[0],
                                      isem0).wait()
                pltpu.async_copy(y_hbm.at[slots.at[0, 0]], rows0, gsem0)
            pl.when(p < npair - 1)(_g2)

            pltpu.make_async_copy(y_hbm.at[slots.at[1, 0]], rows1,
                                  gsem1).wait()
            pltpu.sync_copy(rows1, acc.at[slots.at[1, 1]], add=True)

            def _pre1():
                pltpu.async_copy(e_hbm.at[wid, c0 + 3], slots.at[1], isem1)
                pltpu.make_async_copy(e_hbm.at[wid, c0 + 3], slots.at[1],
                                      isem1).wait()
            pl.when(p < npair - 1)(_pre1)
            return carry

        lax.fori_loop(0, npair, pair, 0)

        plsc.subcore_barrier()
        pltpu.sync_copy(acc.at[pl.ds(s_ax * RT, RT), :],
                        out_hbm.at[c_ax, pl.ds(s_ax * RT, RT), :])

    return agg_kernel


def _prep_edges(src, dst, N, K, NCA, NCB):
    E = src.shape[0]
    NCM = max(NCA, NCB)
    pad = _NS * (NCA + NCB) * K - E
    src_p = jnp.concatenate([src, jnp.zeros((pad,), src.dtype)])
    dst_p = jnp.concatenate([dst, jnp.full((pad,), N, dst.dtype)])

    def per_core(v):
        ea = _NS * NCA * K
        a = v[:ea].reshape(_NS, NCA, K)
        b = v[ea:].reshape(_NS, NCB, K)
        if NCA < NCM:
            a = jnp.pad(a, ((0, 0), (0, NCM - NCA), (0, 0)),
                        constant_values=0)
        if NCB < NCM:
            b = jnp.pad(b, ((0, 0), (0, NCM - NCB), (0, 0)),
                        constant_values=0)
        return jnp.stack([a, b], axis=1).reshape(_NW, NCM, 1, K)

    return jnp.concatenate([per_core(src_p), per_core(dst_p)], axis=2)


def _tc_first(dis, x, W):
    N, F = x.shape
    H = W.shape[1]

    def body(dis_ref, x_ref, w_ref, y_ref):
        xw = jnp.dot(x_ref[...], w_ref[...],
                     preferred_element_type=jnp.float32)
        y_ref[...] = dis_ref[...] * xw

    return pl.pallas_call(
        body,
        out_shape=jax.ShapeDtypeStruct((N, H), jnp.float32),
    )(dis, x, W)


def _tc_mid(sp, y, dis, b, W):
    N, H = y.shape

    def body(sp_ref, y_ref, dis_ref, b_ref, w_ref, o_ref):
        sagg = sp_ref[0][:N] + sp_ref[1][:N] + y_ref[...]
        h = jnp.maximum(dis_ref[...] * sagg + b_ref[...], 0.0)
        hw = jnp.dot(h, w_ref[...],
                     preferred_element_type=jnp.float32)
        o_ref[...] = dis_ref[...] * hw

    return pl.pallas_call(
        body,
        out_shape=jax.ShapeDtypeStruct((N, W.shape[1]), jnp.float32),
    )(sp, y, dis, b, W)


def _tc_final(sp, y, dis, b, batch2d, Wh1, bh1, Wh2, bh2):
    N, H = y.shape

    def body(sp_ref, y_ref, dis_ref, b_ref, bt_ref,
             w1_ref, b1_ref, w2_ref, b2_ref, o_ref):
        h2 = jnp.maximum(
            dis_ref[...] * (sp_ref[0][:N] + sp_ref[1][:N] + y_ref[...])
            + b_ref[...],
            0.0)
        gids = lax.broadcasted_iota(jnp.int32, (_G, 1), 0)
        oh = (gids == bt_ref[...]).astype(jnp.float32)
        cnt = jnp.sum(oh, axis=1, keepdims=True)
        summ = jnp.dot(oh, h2,
                       preferred_element_type=jnp.float32,
                       precision=lax.Precision.HIGHEST)
        pooled = summ / jnp.maximum(cnt, 1.0)
        t = jnp.maximum(
            jnp.dot(pooled, w1_ref[...],
                    preferred_element_type=jnp.float32) + b1_ref[...],
            0.0)
        o_ref[...] = jnp.dot(t, w2_ref[...],
                             preferred_element_type=jnp.float32) + b2_ref[...]

    return pl.pallas_call(
        body,
        out_shape=jax.ShapeDtypeStruct((_G, 1), jnp.float32),
    )(sp, y, dis, b, batch2d, Wh1, bh1, Wh2, bh2)


def kernel(x, edge_index, batch, W1, b1, W2, b2, Wh1, bh1, Wh2, bh2):
    N, F = x.shape
    E = edge_index.shape[1]
    K = 80
    NCA, NCB = 154, 96

    src = edge_index[0]
    dst = edge_index[1]

    dis_mat = _tc_deg(dst.reshape(1, E), dst.reshape(E, 1), N, 6400)
    dis = dis_mat.reshape(-1, 1)[:N]
    y1 = _tc_first(dis, x.astype(jnp.float32), W1)
    z = jnp.zeros((_pad_rows(N), F), jnp.float32)
    e_ab = _prep_edges(src, dst, N, K, NCA, NCB)
    e_ba = _prep_edges(src, dst, N, K, NCB, NCA)
    s1p = _make_agg_pair(N, F, K, NCA, NCB)(y1, e_ab, z)
    y2 = _tc_mid(s1p, y1, dis, b1, W2)
    s2p = _make_agg_pair(N, F, K, NCB, NCA)(y2, e_ba, z)
    out = _tc_final(s2p, y2, dis, b2,
                    batch.reshape(1, N).astype(jnp.int32),
                    Wh1, bh1, Wh2, bh2)
    return out


# speedup vs baseline: 2.8997x; 1.3630x over previous
import functools

import jax
import jax.numpy as jnp
from jax import lax
from jax.experimental import pallas as pl
from jax.experimental.pallas import tpu as pltpu
from jax.experimental.pallas import tpu_sc as plsc

_NC = 2
_NS = 16
_NW = _NC * _NS
_L = 16
_G = 64


def _mesh():
    return plsc.VectorSubcoreMesh(core_axis_name="c", subcore_axis_name="s",
                                  num_cores=_NC, num_subcores=_NS)


def _pad_rows(N):
    return -(-N // (8 * _NS)) * (8 * _NS)


def _tc_deg(dst_row, N, EB):
    E = dst_row.shape[1]
    HI = -(-N // 128)
    grid = E // EB

    def body(dr_ref, o_ref):
        i = pl.program_id(0)

        @pl.when(i == 0)
        def _init():
            o_ref[...] = jnp.zeros_like(o_ref)

        d = dr_ref[...]
        oh_hi = (lax.broadcasted_iota(jnp.int32, (HI, 1), 0) == d // 128
                 ).astype(jnp.float32)
        oh_lo = (lax.broadcasted_iota(jnp.int32, (128, 1), 0) == d % 128
                 ).astype(jnp.float32)
        o_ref[...] += lax.dot_general(
            oh_lo, oh_hi, (((1,), (1,)), ((), ())),
            preferred_element_type=jnp.float32)

        @pl.when(i == grid - 1)
        def _finish():
            o_ref[...] = lax.rsqrt(o_ref[...] + 1.0)

    return pl.pallas_call(
        body,
        grid=(grid,),
        in_specs=[pl.BlockSpec((1, EB), lambda i: (0, i))],
        out_specs=pl.BlockSpec((128, HI), lambda i: (0, 0)),
        out_shape=jax.ShapeDtypeStruct((128, HI), jnp.float32),
    )(dst_row)


@functools.lru_cache(maxsize=None)
def _make_agg_kernel(N, F, K, NCHUNK):
    NP = _pad_rows(N)
    RT = NP // _NS
    NB, NQ = 4, 8
    NG = NCHUNK // NQ
    assert NCHUNK % NQ == 0

    @functools.partial(
        pl.kernel,
        out_type=jax.ShapeDtypeStruct((_NC, NP, F), jnp.float32),
        mesh=_mesh(),
        scratch_types=(
            [pltpu.VMEM_SHARED((NP, F), jnp.float32)]
            + [pltpu.VMEM((K, F), jnp.float32) for _ in range(NB)]
            + [pltpu.VMEM((NQ, 2, K), jnp.int32)]
            + [pltpu.SemaphoreType.DMA] * (2 * NB + NQ)
        ),
    )
    def agg_kernel(y_hbm, e_hbm, z_hbm, out_hbm, acc, *rest):
        rows = rest[:NB]
        slots = rest[NB]
        gsem = rest[NB + 1: NB + 1 + NB]
        ssem = rest[NB + 1 + NB: NB + 1 + 2 * NB]
        isem = rest[NB + 1 + 2 * NB:]
        c_ax = lax.axis_index("c")
        s_ax = lax.axis_index("s")
        wid = s_ax * _NC + c_ax

        pltpu.sync_copy(z_hbm.at[pl.ds(s_ax * RT, RT), :],
                        acc.at[pl.ds(s_ax * RT, RT), :])
        plsc.subcore_barrier()

        def idx_load(ci, q):
            return pltpu.async_copy(e_hbm.at[wid, ci], slots.at[q], isem[q])

        def gather(q, b):
            return pltpu.async_copy(y_hbm.at[slots.at[q, 0]], rows[b], gsem[b])

        def scatter(q, b):
            return pltpu.async_copy(rows[b], acc.at[slots.at[q, 1]], ssem[b],
                                    add=True)

        for q in range(NQ):
            idx_load(q, q)
        pltpu.make_async_copy(e_hbm.at[wid, 0], slots.at[0], isem[0]).wait()
        gather(0, 0)
        pltpu.make_async_copy(e_hbm.at[wid, 1], slots.at[1], isem[1]).wait()
        gather(1, 1)

        def group(g, carry):
            for k in range(NQ):
                b, q = k % NB, k
                b2, q2 = (k - 2) % NB, (k - 2) % NQ
                pltpu.make_async_copy(y_hbm.at[slots.at[q, 0]], rows[b],
                                      gsem[b]).wait()
                scatter(q, b)

                def _wait_sc():
                    pltpu.make_async_copy(rows[b2], acc.at[slots.at[q2, 1]],
                                          ssem[b2]).wait()
                if k >= 2:
                    _wait_sc()
                else:
                    pl.when(g > 0)(_wait_sc)

                def _iload():
                    idx_load(g * NQ + k + 6, q2)
                if k >= 2:
                    pl.when(g * NQ + k + 6 < NCHUNK)(_iload)
                else:
                    pl.when(jnp.logical_and(
                        g > 0, g * NQ + k + 6 < NCHUNK))(_iload)

                q3 = (k + 2) % NQ
                def _gath():
                    pltpu.make_async_copy(e_hbm.at[wid, g * NQ + k + 2],
                                          slots.at[q3], isem[q3]).wait()
                    gather(q3, b2)
                if k < NQ - 2:
                    _gath()
                else:
                    pl.when(g < NG - 1)(_gath)
            return carry

        lax.fori_loop(0, NG, group, 0)

        for k in (NQ - 2, NQ - 1):
            pltpu.make_async_copy(rows[k % NB], acc.at[slots.at[k, 1]],
                                  ssem[k % NB]).wait()

        plsc.subcore_barrier()
        pltpu.sync_copy(acc.at[pl.ds(s_ax * RT, RT), :],
                        out_hbm.at[c_ax, pl.ds(s_ax * RT, RT), :])

    return agg_kernel


@functools.lru_cache(maxsize=None)
def _make_agg_serial(N, F, K, NCHUNK):
    NP = _pad_rows(N)
    RT = NP // _NS

    @functools.partial(
        pl.kernel,
        out_type=jax.ShapeDtypeStruct((_NC, NP, F), jnp.float32),
        mesh=_mesh(),
        scratch_types=[
            pltpu.VMEM_SHARED((NP, F), jnp.float32),
            pltpu.VMEM((2, K), jnp.int32),
            pltpu.VMEM((K, F), jnp.float32),
            pltpu.SemaphoreType.DMA,
        ],
    )
    def agg_kernel(y_hbm, e_hbm, z_hbm, out_hbm, acc, slot, rows, sem):
        c_ax = lax.axis_index("c")
        s_ax = lax.axis_index("s")
        wid = s_ax * _NC + c_ax

        pltpu.sync_copy(z_hbm.at[pl.ds(s_ax * RT, RT), :],
                        acc.at[pl.ds(s_ax * RT, RT), :])
        plsc.subcore_barrier()

        def chunk(ci, carry):
            pltpu.sync_copy(e_hbm.at[wid, ci], slot)
            pltpu.async_copy(y_hbm.at[slot.at[0]], rows, sem).wait()
            pltpu.sync_copy(rows, acc.at[slot.at[1]], add=True)
            return carry
        lax.fori_loop(0, NCHUNK, chunk, 0)

        plsc.subcore_barrier()
        pltpu.sync_copy(acc.at[pl.ds(s_ax * RT, RT), :],
                        out_hbm.at[c_ax, pl.ds(s_ax * RT, RT), :])

    return agg_kernel


@functools.lru_cache(maxsize=None)
def _make_agg_pair(N, F, K, NCA, NCB):
    NP = _pad_rows(N)
    RT = NP // _NS
    NCHUNK = max(NCA, NCB)
    assert NCA % 2 == 0 and NCB % 2 == 0

    @functools.partial(
        pl.kernel,
        out_type=jax.ShapeDtypeStruct((_NC, NP, F), jnp.float32),
        mesh=_mesh(),
        scratch_types=[
            pltpu.VMEM_SHARED((NP, F), jnp.float32),
            pltpu.VMEM((2, 2, K), jnp.int32),
            pltpu.VMEM((K, F), jnp.float32),
            pltpu.VMEM((K, F), jnp.float32),
            pltpu.SemaphoreType.DMA,
            pltpu.SemaphoreType.DMA,
            pltpu.SemaphoreType.DMA,
            pltpu.SemaphoreType.DMA,
        ],
    )
    def agg_kernel(y_hbm, e_hbm, z_hbm, out_hbm,
                   acc, slots, rows0, rows1, gsem0, gsem1, isem0, isem1):
        c_ax = lax.axis_index("c")
        s_ax = lax.axis_index("s")
        wid = s_ax * _NC + c_ax

        npair = jnp.where(c_ax == 0, NCA // 2, NCB // 2)

        pltpu.sync_copy(z_hbm.at[pl.ds(s_ax * RT, RT), :],
                        acc.at[pl.ds(s_ax * RT, RT), :])
        plsc.subcore_barrier()

        pltpu.sync_copy(e_hbm.at[wid, 0], slots.at[0])
        pltpu.sync_copy(e_hbm.at[wid, 1], slots.at[1])
        pltpu.async_copy(y_hbm.at[slots.at[0, 0]], rows0, gsem0)

        def pair(p, carry):
            c0 = 2 * p
            pltpu.make_async_copy(y_hbm.at[slots.at[0, 0]], rows0,
                                  gsem0).wait()
            pltpu.async_copy(y_hbm.at[slots.at[1, 0]], rows1, gsem1)
            pltpu.sync_copy(rows0, acc.at[slots.at[0, 1]], add=True)

            def _pre0():
                pltpu.async_copy(e_hbm.at[wid, c0 + 2], slots.at[0], isem0)
            pl.when(p < npair - 1)(_pre0)

            def _g2():
                pltpu.make_async_copy(e_hbm.at[wid, c0 + 2], slots.at[0],
                                      isem0).wait()
                pltpu.async_copy(y_hbm.at[slots.at[0, 0]], rows0, gsem0)
            pl.when(p < npair - 1)(_g2)

            pltpu.make_async_copy(y_hbm.at[slots.at[1, 0]], rows1,
                                  gsem1).wait()
            pltpu.sync_copy(rows1, acc.at[slots.at[1, 1]], add=True)

            def _pre1():
                pltpu.async_copy(e_hbm.at[wid, c0 + 3], slots.at[1], isem1)
                pltpu.make_async_copy(e_hbm.at[wid, c0 + 3], slots.at[1],
                                      isem1).wait()
            pl.when(p < npair - 1)(_pre1)
            return carry

        lax.fori_loop(0, npair, pair, 0)

        plsc.subcore_barrier()
        pltpu.sync_copy(acc.at[pl.ds(s_ax * RT, RT), :],
                        out_hbm.at[c_ax, pl.ds(s_ax * RT, RT), :])

    return agg_kernel


def _prep_edges(src, dst, N, K, NCA, NCB):
    E = src.shape[0]
    NCM = max(NCA, NCB)
    pad = _NS * (NCA + NCB) * K - E
    src_p = jnp.concatenate([src, jnp.zeros((pad,), src.dtype)])
    dst_p = jnp.concatenate([dst, jnp.full((pad,), N, dst.dtype)])

    def per_core(v):
        ea = _NS * NCA * K
        a = v[:ea].reshape(_NS, NCA, K)
        b = v[ea:].reshape(_NS, NCB, K)
        if NCA < NCM:
            a = jnp.pad(a, ((0, 0), (0, NCM - NCA), (0, 0)),
                        constant_values=0)
        if NCB < NCM:
            b = jnp.pad(b, ((0, 0), (0, NCM - NCB), (0, 0)),
                        constant_values=0)
        return jnp.stack([a, b], axis=1).reshape(_NW, NCM, 1, K)

    return jnp.concatenate([per_core(src_p), per_core(dst_p)], axis=2)


def _tc_first(dis, x, W):
    N, F = x.shape
    H = W.shape[1]

    def body(dis_ref, x_ref, w_ref, y_ref):
        xw = jnp.dot(x_ref[...], w_ref[...],
                     preferred_element_type=jnp.float32)
        y_ref[...] = dis_ref[...] * xw

    return pl.pallas_call(
        body,
        out_shape=jax.ShapeDtypeStruct((N, H), jnp.float32),
    )(dis, x, W)


def _tc_mid(sp, y, dis, b, W):
    N, H = y.shape

    def body(sp_ref, y_ref, dis_ref, b_ref, w_ref, o_ref):
        sagg = sp_ref[0][:N] + sp_ref[1][:N] + y_ref[...]
        h = jnp.maximum(dis_ref[...] * sagg + b_ref[...], 0.0)
        hw = jnp.dot(h, w_ref[...],
                     preferred_element_type=jnp.float32)
        o_ref[...] = dis_ref[...] * hw

    return pl.pallas_call(
        body,
        out_shape=jax.ShapeDtypeStruct((N, W.shape[1]), jnp.float32),
    )(sp, y, dis, b, W)


def _tc_final(sp, y, dis, b, batch2d, Wh1, bh1, Wh2, bh2):
    N, H = y.shape

    def body(sp_ref, y_ref, dis_ref, b_ref, bt_ref,
             w1_ref, b1_ref, w2_ref, b2_ref, o_ref):
        h2 = jnp.maximum(
            dis_ref[...] * (sp_ref[0][:N] + sp_ref[1][:N] + y_ref[...])
            + b_ref[...],
            0.0)
        gids = lax.broadcasted_iota(jnp.int32, (_G, 1), 0)
        oh = (gids == bt_ref[...]).astype(jnp.float32)
        cnt = jnp.sum(oh, axis=1, keepdims=True)
        summ = jnp.dot(oh, h2,
                       preferred_element_type=jnp.float32,
                       precision=lax.Precision.HIGHEST)
        pooled = summ / jnp.maximum(cnt, 1.0)
        t = jnp.maximum(
            jnp.dot(pooled, w1_ref[...],
                    preferred_element_type=jnp.float32) + b1_ref[...],
            0.0)
        o_ref[...] = jnp.dot(t, w2_ref[...],
                             preferred_element_type=jnp.float32) + b2_ref[...]

    return pl.pallas_call(
        body,
        out_shape=jax.ShapeDtypeStruct((_G, 1), jnp.float32),
    )(sp, y, dis, b, batch2d, Wh1, bh1, Wh2, bh2)


def kernel(x, edge_index, batch, W1, b1, W2, b2, Wh1, bh1, Wh2, bh2):
    N, F = x.shape
    E = edge_index.shape[1]
    K = 80

    src = edge_index[0]
    dst = edge_index[1]

    dis_matT = _tc_deg(dst.reshape(1, E), N, 6400)
    dis = dis_matT.T.reshape(-1, 1)[:N]
    y1 = _tc_first(dis, x.astype(jnp.float32), W1)
    z = jnp.zeros((_pad_rows(N), F), jnp.float32)
    e_a = _prep_edges(src, dst, N, K, 126, 124)
    e_b = _prep_edges(src, dst, N, K, 140, 110)
    s1p = _make_agg_pair(N, F, K, 126, 124)(y1, e_a, z)
    y2 = _tc_mid(s1p, y1, dis, b1, W2)
    s2p = _make_agg_pair(N, F, K, 140, 110)(y2, e_b, z)
    out = _tc_final(s2p, y2, dis, b2,
                    batch.reshape(1, N).astype(jnp.int32),
                    Wh1, bh1, Wh2, bh2)
    return out


# speedup vs baseline: 3.0817x; 1.0628x over previous
import functools

import jax
import jax.numpy as jnp
from jax import lax
from jax.experimental import pallas as pl
from jax.experimental.pallas import tpu as pltpu
from jax.experimental.pallas import tpu_sc as plsc

_NC = 2
_NS = 16
_NW = _NC * _NS
_L = 16
_G = 64


def _mesh():
    return plsc.VectorSubcoreMesh(core_axis_name="c", subcore_axis_name="s",
                                  num_cores=_NC, num_subcores=_NS)


def _pad_rows(N):
    return -(-N // (8 * _NS)) * (8 * _NS)


def _tc_deg(dst_row, N, EB):
    E = dst_row.shape[1]
    HI = -(-N // 128)
    grid = E // EB

    def body(dr_ref, o_ref):
        i = pl.program_id(0)

        @pl.when(i == 0)
        def _init():
            o_ref[...] = jnp.zeros_like(o_ref)

        d = dr_ref[...]
        oh_hi = (lax.broadcasted_iota(jnp.int32, (HI, 1), 0) == d // 128
                 ).astype(jnp.float32)
        oh_lo = (lax.broadcasted_iota(jnp.int32, (128, 1), 0) == d % 128
                 ).astype(jnp.float32)
        o_ref[...] += lax.dot_general(
            oh_lo, oh_hi, (((1,), (1,)), ((), ())),
            preferred_element_type=jnp.float32)

        @pl.when(i == grid - 1)
        def _finish():
            o_ref[...] = lax.rsqrt(o_ref[...] + 1.0)

    return pl.pallas_call(
        body,
        grid=(grid,),
        in_specs=[pl.BlockSpec((1, EB), lambda i: (0, i))],
        out_specs=pl.BlockSpec((128, HI), lambda i: (0, 0)),
        out_shape=jax.ShapeDtypeStruct((128, HI), jnp.float32),
    )(dst_row)


@functools.lru_cache(maxsize=None)
def _make_agg_kernel(N, F, K, NCHUNK):
    NP = _pad_rows(N)
    RT = NP // _NS
    NB, NQ = 4, 8
    NG = NCHUNK // NQ
    assert NCHUNK % NQ == 0

    @functools.partial(
        pl.kernel,
        out_type=jax.ShapeDtypeStruct((_NC, NP, F), jnp.float32),
        mesh=_mesh(),
        scratch_types=(
            [pltpu.VMEM_SHARED((NP, F), jnp.float32)]
            + [pltpu.VMEM((K, F), jnp.float32) for _ in range(NB)]
            + [pltpu.VMEM((NQ, 2, K), jnp.int32)]
            + [pltpu.SemaphoreType.DMA] * (2 * NB + NQ)
        ),
    )
    def agg_kernel(y_hbm, e_hbm, z_hbm, out_hbm, acc, *rest):
        rows = rest[:NB]
        slots = rest[NB]
        gsem = rest[NB + 1: NB + 1 + NB]
        ssem = rest[NB + 1 + NB: NB + 1 + 2 * NB]
        isem = rest[NB + 1 + 2 * NB:]
        c_ax = lax.axis_index("c")
        s_ax = lax.axis_index("s")
        wid = s_ax * _NC + c_ax

        pltpu.sync_copy(z_hbm.at[pl.ds(s_ax * RT, RT), :],
                        acc.at[pl.ds(s_ax * RT, RT), :])
        plsc.subcore_barrier()

        def idx_load(ci, q):
            return pltpu.async_copy(e_hbm.at[wid, ci], slots.at[q], isem[q])

        def gather(q, b):
            return pltpu.async_copy(y_hbm.at[slots.at[q, 0]], rows[b], gsem[b])

        def scatter(q, b):
            return pltpu.async_copy(rows[b], acc.at[slots.at[q, 1]], ssem[b],
                                    add=True)

        for q in range(NQ):
            idx_load(q, q)
        pltpu.make_async_copy(e_hbm.at[wid, 0], slots.at[0], isem[0]).wait()
        gather(0, 0)
        pltpu.make_async_copy(e_hbm.at[wid, 1], slots.at[1], isem[1]).wait()
        gather(1, 1)

        def group(g, carry):
            for k in range(NQ):
                b, q = k % NB, k
                b2, q2 = (k - 2) % NB, (k - 2) % NQ
                pltpu.make_async_copy(y_hbm.at[slots.at[q, 0]], rows[b],
                                      gsem[b]).wait()
                scatter(q, b)

                def _wait_sc():
                    pltpu.make_async_copy(rows[b2], acc.at[slots.at[q2, 1]],
                                          ssem[b2]).wait()
                if k >= 2:
                    _wait_sc()
                else:
                    pl.when(g > 0)(_wait_sc)

                def _iload():
                    idx_load(g * NQ + k + 6, q2)
                if k >= 2:
                    pl.when(g * NQ + k + 6 < NCHUNK)(_iload)
                else:
                    pl.when(jnp.logical_and(
                        g > 0, g * NQ + k + 6 < NCHUNK))(_iload)

                q3 = (k + 2) % NQ
                def _gath():
                    pltpu.make_async_copy(e_hbm.at[wid, g * NQ + k + 2],
                                          slots.at[q3], isem[q3]).wait()
                    gather(q3, b2)
                if k < NQ - 2:
                    _gath()
                else:
                    pl.when(g < NG - 1)(_gath)
            return carry

        lax.fori_loop(0, NG, group, 0)

        for k in (NQ - 2, NQ - 1):
            pltpu.make_async_copy(rows[k % NB], acc.at[slots.at[k, 1]],
                                  ssem[k % NB]).wait()

        plsc.subcore_barrier()
        pltpu.sync_copy(acc.at[pl.ds(s_ax * RT, RT), :],
                        out_hbm.at[c_ax, pl.ds(s_ax * RT, RT), :])

    return agg_kernel


@functools.lru_cache(maxsize=None)
def _make_agg_serial(N, F, K, NCHUNK):
    NP = _pad_rows(N)
    RT = NP // _NS

    @functools.partial(
        pl.kernel,
        out_type=jax.ShapeDtypeStruct((_NC, NP, F), jnp.float32),
        mesh=_mesh(),
        scratch_types=[
            pltpu.VMEM_SHARED((NP, F), jnp.float32),
            pltpu.VMEM((2, K), jnp.int32),
            pltpu.VMEM((K, F), jnp.float32),
            pltpu.SemaphoreType.DMA,
        ],
    )
    def agg_kernel(y_hbm, e_hbm, z_hbm, out_hbm, acc, slot, rows, sem):
        c_ax = lax.axis_index("c")
        s_ax = lax.axis_index("s")
        wid = s_ax * _NC + c_ax

        pltpu.sync_copy(z_hbm.at[pl.ds(s_ax * RT, RT), :],
                        acc.at[pl.ds(s_ax * RT, RT), :])
        plsc.subcore_barrier()

        def chunk(ci, carry):
            pltpu.sync_copy(e_hbm.at[wid, ci], slot)
            pltpu.async_copy(y_hbm.at[slot.at[0]], rows, sem).wait()
            pltpu.sync_copy(rows, acc.at[slot.at[1]], add=True)
            return carry
        lax.fori_loop(0, NCHUNK, chunk, 0)

        plsc.subcore_barrier()
        pltpu.sync_copy(acc.at[pl.ds(s_ax * RT, RT), :],
                        out_hbm.at[c_ax, pl.ds(s_ax * RT, RT), :])

    return agg_kernel


@functools.lru_cache(maxsize=None)
def _make_agg_pair(N, F, K, NCA, NCB):
    NP = _pad_rows(N)
    RT = NP // _NS
    NCHUNK = max(NCA, NCB)
    assert NCA % 2 == 0 and NCB % 2 == 0

    @functools.partial(
        pl.kernel,
        out_type=jax.ShapeDtypeStruct((_NC, NP, F), jnp.float32),
        mesh=_mesh(),
        scratch_types=[
            pltpu.VMEM_SHARED((NP, F), jnp.float32),
            pltpu.VMEM((2, 2, K), jnp.int32),
            pltpu.VMEM((K, F), jnp.float32),
            pltpu.VMEM((K, F), jnp.float32),
            pltpu.SemaphoreType.DMA,
            pltpu.SemaphoreType.DMA,
            pltpu.SemaphoreType.DMA,
            pltpu.SemaphoreType.DMA,
        ],
    )
    def agg_kernel(y_hbm, e_hbm, z_hbm, out_hbm,
                   acc, slots, rows0, rows1, gsem0, gsem1, isem0, isem1):
        c_ax = lax.axis_index("c")
        s_ax = lax.axis_index("s")
        wid = s_ax * _NC + c_ax

        npair = jnp.where(c_ax == 0, NCA // 2, NCB // 2)

        pltpu.sync_copy(z_hbm.at[pl.ds(s_ax * RT, RT), :],
                        acc.at[pl.ds(s_ax * RT, RT), :])
        plsc.subcore_barrier()

        pltpu.sync_copy(e_hbm.at[wid, 0], slots.at[0])
        pltpu.sync_copy(e_hbm.at[wid, 1], slots.at[1])
        pltpu.async_copy(y_hbm.at[slots.at[0, 0]], rows0, gsem0)

        def pair(p, carry):
            c0 = 2 * p
            pltpu.make_async_copy(y_hbm.at[slots.at[0, 0]], rows0,
                                  gsem0).wait()
            pltpu.async_copy(y_hbm.at[slots.at[1, 0]], rows1, gsem1)
            pltpu.sync_copy(rows0, acc.at[slots.at[0, 1]], add=True)

            def _pre0():
                pltpu.async_copy(e_hbm.at[wid, c0 + 2], slots.at[0], isem0)
            pl.when(p < npair - 1)(_pre0)

            def _g2():
                pltpu.make_async_copy(e_hbm.at[wid, c0 + 2], slots.at[0],
                                      isem0).wait()
                pltpu.async_copy(y_hbm.at[slots.at[0, 0]], rows0, gsem0)
            pl.when(p < npair - 1)(_g2)

            pltpu.make_async_copy(y_hbm.at[slots.at[1, 0]], rows1,
                                  gsem1).wait()
            pltpu.sync_copy(rows1, acc.at[slots.at[1, 1]], add=True)

            def _pre1():
                pltpu.async_copy(e_hbm.at[wid, c0 + 3], slots.at[1], isem1)
                pltpu.make_async_copy(e_hbm.at[wid, c0 + 3], slots.at[1],
                                      isem1).wait()
            pl.when(p < npair - 1)(_pre1)
            return carry

        lax.fori_loop(0, npair, pair, 0)

        plsc.subcore_barrier()
        pltpu.sync_copy(acc.at[pl.ds(s_ax * RT, RT), :],
                        out_hbm.at[c_ax, pl.ds(s_ax * RT, RT), :])

    return agg_kernel


def _prep_edges(src, dst, N, K, NCA, NCB):
    E = src.shape[0]
    NCM = max(NCA, NCB)
    pad = _NS * (NCA + NCB) * K - E
    src_p = jnp.concatenate([src, jnp.zeros((pad,), src.dtype)])
    dst_p = jnp.concatenate([dst, jnp.full((pad,), N, dst.dtype)])

    def per_core(v):
        ea = _NS * NCA * K
        a = v[:ea].reshape(_NS, NCA, K)
        b = v[ea:].reshape(_NS, NCB, K)
        if NCA < NCM:
            a = jnp.pad(a, ((0, 0), (0, NCM - NCA), (0, 0)),
                        constant_values=0)
        if NCB < NCM:
            b = jnp.pad(b, ((0, 0), (0, NCM - NCB), (0, 0)),
                        constant_values=0)
        return jnp.stack([a, b], axis=1).reshape(_NW, NCM, 1, K)

    return jnp.concatenate([per_core(src_p), per_core(dst_p)], axis=2)


def _tc_first(dis, x, W):
    N, F = x.shape
    H = W.shape[1]

    def body(dis_ref, x_ref, w_ref, y_ref):
        xw = jnp.dot(x_ref[...], w_ref[...],
                     preferred_element_type=jnp.float32)
        y_ref[...] = dis_ref[...] * xw

    return pl.pallas_call(
        body,
        out_shape=jax.ShapeDtypeStruct((N, H), jnp.float32),
    )(dis, x, W)


def _tc_mid(sp, y, dis, b, W):
    N, H = y.shape

    def body(sp_ref, y_ref, dis_ref, b_ref, w_ref, o_ref):
        sagg = sp_ref[0][:N] + sp_ref[1][:N] + y_ref[...]
        h = jnp.maximum(dis_ref[...] * sagg + b_ref[...], 0.0)
        hw = jnp.dot(h, w_ref[...],
                     preferred_element_type=jnp.float32)
        o_ref[...] = dis_ref[...] * hw

    return pl.pallas_call(
        body,
        out_shape=jax.ShapeDtypeStruct((N, W.shape[1]), jnp.float32),
    )(sp, y, dis, b, W)


def _tc_final(sp, y, dis, b, batch2d, Wh1, bh1, Wh2, bh2):
    N, H = y.shape

    def body(sp_ref, y_ref, dis_ref, b_ref, bt_ref,
             w1_ref, b1_ref, w2_ref, b2_ref, o_ref):
        h2 = jnp.maximum(
            dis_ref[...] * (sp_ref[0][:N] + sp_ref[1][:N] + y_ref[...])
            + b_ref[...],
            0.0)
        gids = lax.broadcasted_iota(jnp.int32, (_G, 1), 0)
        oh = (gids == bt_ref[...]).astype(jnp.float32)
        cnt = jnp.sum(oh, axis=1, keepdims=True)
        summ = jnp.dot(oh, h2,
                       preferred_element_type=jnp.float32,
                       precision=lax.Precision.HIGHEST)
        pooled = summ / jnp.maximum(cnt, 1.0)
        t = jnp.maximum(
            jnp.dot(pooled, w1_ref[...],
                    preferred_element_type=jnp.float32) + b1_ref[...],
            0.0)
        o_ref[...] = jnp.dot(t, w2_ref[...],
                             preferred_element_type=jnp.float32) + b2_ref[...]

    return pl.pallas_call(
        body,
        out_shape=jax.ShapeDtypeStruct((_G, 1), jnp.float32),
    )(sp, y, dis, b, batch2d, Wh1, bh1, Wh2, bh2)


def kernel(x, edge_index, batch, W1, b1, W2, b2, Wh1, bh1, Wh2, bh2):
    N, F = x.shape
    E = edge_index.shape[1]
    K = 80

    src = edge_index[0]
    dst = edge_index[1]

    dis_matT = _tc_deg(dst.reshape(1, E), N, 12800)
    dis = dis_matT.T.reshape(-1, 1)[:N]
    y1 = _tc_first(dis, x.astype(jnp.float32), W1)
    z = jnp.zeros((_pad_rows(N), F), jnp.float32)
    e_il = _prep_edges(src, dst, N, K, 126, 124)
    agg = _make_agg_pair(N, F, K, 126, 124)
    s1p = agg(y1, e_il, z)
    y2 = _tc_mid(s1p, y1, dis, b1, W2)
    s2p = agg(y2, e_il, z)
    out = _tc_final(s2p, y2, dis, b2,
                    batch.reshape(1, N).astype(jnp.int32),
                    Wh1, bh1, Wh2, bh2)
    return out
